# baseline (speedup 1.0000x reference)
import functools

import jax
import jax.numpy as jnp
from jax import lax
from jax.experimental import pallas as pl
from jax.experimental.pallas import tpu as pltpu
from jax.experimental.pallas import tpu_sc as plsc

_D = 123
_DP = 128
_NW = 32
_CH = 400
_XR = 2
_NBUF = 2


def _fill_table_row(tab_v, emb_v, v, off, sh):
    ev = emb_v[pl.ds(off, 16)]
    h, vol, ch = ev[sh], ev[sh + 1], ev[sh + 2]
    p3, p4, p5 = ev[sh + 3], ev[sh + 4], ev[sh + 5]
    for g in range(8):
        l = lax.iota(jnp.int32, 16) + g * 16
        lf = l.astype(jnp.float32)
        in0 = l < 90
        in1 = l < 112
        in2 = l < 120
        mu = jnp.where(
            in0,
            -4.5 + lf * (9.0 / 89.0),
            jnp.where(
                in1,
                (lf - 90.0) * (2.2 / 21.0),
                -1.0 + (lf - 112.0) * (2.0 / 7.0),
            ),
        )
        val = jnp.where(in0, h, jnp.where(in1, vol, ch))
        inv = jnp.where(in1, 10.0, 4.0)
        d = (val - mu) * inv
        r = jnp.exp(-(d * d))
        pv = jnp.where(l == 120, p3, jnp.where(l == 121, p4, p5))
        s = 1.0 / (1.0 + jnp.exp(3.0 - 6.0 * pv))
        tab_v[v, pl.ds(g * 16, 16)] = jnp.where(in2, r, s)


def _build_sc_gather(xb, xl):
    n_rows = xb * xl
    rows_per_w = n_rows // _NW
    xrows_per_w = xb // _NW
    n_chunks = rows_per_w // _CH
    mesh = plsc.VectorSubcoreMesh(core_axis_name="c", subcore_axis_name="s")

    @functools.partial(
        pl.kernel,
        mesh=mesh,
        out_type=jax.ShapeDtypeStruct((n_rows, _D), jnp.float32),
        scratch_types=(
            [pltpu.VMEM((21, _DP), jnp.float32)]
            + [pltpu.VMEM((126,), jnp.float32)]
            + [pltpu.VMEM((_XR, 200), jnp.int32) for _ in range(_NBUF)]
            + [pltpu.VMEM((_CH, _D), jnp.float32) for _ in range(_NBUF)]
            + [pltpu.SemaphoreType.DMA for _ in range(2 * _NBUF)]
        ),
    )
    def sc_gather(x_hbm, emb_hbm, out_hbm, tab_v, emb_v, *bufs_flat):
        idxs = bufs_flat[:_NBUF]
        outs = bufs_flat[_NBUF : 2 * _NBUF]
        osems = bufs_flat[2 * _NBUF : 3 * _NBUF]
        isems = bufs_flat[3 * _NBUF :]
        wid = lax.axis_index("s") * 2 + lax.axis_index("c")
        base = wid * rows_per_w
        xbase = wid * xrows_per_w
        n_rounds = n_chunks // _NBUF

        for k in range(_NBUF - 1):
            pltpu.async_copy(
                x_hbm.at[pl.ds(xbase + k * _XR, _XR)], idxs[k], isems[k]
            )

        pltpu.sync_copy(emb_hbm, emb_v)

        @plsc.parallel_loop(0, 19)
        def tab_row(v):
            _fill_table_row(tab_v, emb_v, v, v * 6, 0)

        _fill_table_row(tab_v, emb_v, 19, 110, 4)
        _fill_table_row(tab_v, emb_v, 20, 110, 10)

        def rnd(p, carry):
            for k in range(_NBUF):
                idx_v, out_v, osem, isem = idxs[k], outs[k], osems[k], isems[k]
                c = p * _NBUF + k
                b0 = base + c * _CH
                xr0 = xbase + c * _XR

                pltpu.make_async_copy(
                    x_hbm.at[pl.ds(xr0, _XR)], idx_v, isem
                ).wait()

                pk = (k + _NBUF - 1) % _NBUF

                @pl.when(c + _NBUF - 1 < n_chunks)
                def _prefetch():
                    nxr0 = xbase + (c + _NBUF - 1) * _XR
                    pltpu.async_copy(
                        x_hbm.at[pl.ds(nxr0, _XR)], idxs[pk], isems[pk]
                    )

                @pl.when(p > 0)
                def _drain():
                    pltpu.make_async_copy(
                        out_v, out_hbm.at[pl.ds(b0, _CH)], osem
                    ).wait()

                def copy_grp(off, r):
                    bv = idx_v[r, pl.ds(off, 16)]
                    for j in range(16):
                        b = bv[j]
                        i = r * 200 + off + j
                        for o in (0, 16, 32, 48, 64, 80, 96, 107):
                            out_v[i, pl.ds(o, 16)] = tab_v[b, pl.ds(o, 16)]

                @plsc.parallel_loop(0, 12)
                def grp(g):
                    off = pl.multiple_of(g * 16, 16)
                    for r in range(_XR):
                        copy_grp(off, r)

                for r in range(_XR):
                    copy_grp(184, r)

                pltpu.async_copy(out_v, out_hbm.at[pl.ds(b0, _CH)], osem)
            return carry

        lax.fori_loop(0, n_rounds, rnd, 0)
        for k in range(_NBUF):
            pltpu.make_async_copy(
                outs[k], out_hbm.at[pl.ds(base, _CH)], osems[k]
            ).wait()

    return sc_gather


def kernel(x, embedding):
    emb_flat = embedding.reshape(-1)
    return _build_sc_gather(x.shape[0], x.shape[1])(x, emb_flat)


# speedup vs baseline: 1.5868x; 1.5868x over previous
import functools

import jax
import jax.numpy as jnp
from jax import lax
from jax.experimental import pallas as pl
from jax.experimental.pallas import tpu as pltpu
from jax.experimental.pallas import tpu_sc as plsc

_D = 123
_DP = 128
_NW = 32
_CH = 320
_NBUF = 2


def _fill_table_row(tab_v, emb_v, v, off, sh):
    ev = emb_v[pl.ds(off, 16)]
    h, vol, ch = ev[sh], ev[sh + 1], ev[sh + 2]
    p3, p4, p5 = ev[sh + 3], ev[sh + 4], ev[sh + 5]
    for g in range(8):
        l = lax.iota(jnp.int32, 16) + g * 16
        lf = l.astype(jnp.float32)
        in0 = l < 90
        in1 = l < 112
        in2 = l < 120
        mu = jnp.where(
            in0,
            -4.5 + lf * (9.0 / 89.0),
            jnp.where(
                in1,
                (lf - 90.0) * (2.2 / 21.0),
                -1.0 + (lf - 112.0) * (2.0 / 7.0),
            ),
        )
        val = jnp.where(in0, h, jnp.where(in1, vol, ch))
        inv = jnp.where(in1, 10.0, 4.0)
        d = (val - mu) * inv
        r = jnp.exp(-(d * d))
        pv = jnp.where(l == 120, p3, jnp.where(l == 121, p4, p5))
        s = 1.0 / (1.0 + jnp.exp(3.0 - 6.0 * pv))
        tab_v[v, pl.ds(g * 16, 16)] = jnp.where(in2, r, s)


def _build_sc_gather(n_rows):
    rows_per_w = n_rows // _NW
    n_chunks = rows_per_w // _CH
    mesh = plsc.VectorSubcoreMesh(core_axis_name="c", subcore_axis_name="s")

    @functools.partial(
        pl.kernel,
        mesh=mesh,
        out_type=jax.ShapeDtypeStruct((n_rows, _D), jnp.float32),
        scratch_types=(
            [pltpu.VMEM((21, _DP), jnp.float32)]
            + [pltpu.VMEM((126,), jnp.float32)]
            + [pltpu.VMEM((_CH,), jnp.int32) for _ in range(_NBUF)]
            + [pltpu.VMEM((_CH, _D), jnp.float32) for _ in range(_NBUF)]
            + [pltpu.SemaphoreType.DMA for _ in range(2 * _NBUF)]
        ),
    )
    def sc_gather(x_hbm, emb_hbm, out_hbm, tab_v, emb_v, *bufs_flat):
        idxs = bufs_flat[:_NBUF]
        outs = bufs_flat[_NBUF : 2 * _NBUF]
        osems = bufs_flat[2 * _NBUF : 3 * _NBUF]
        isems = bufs_flat[3 * _NBUF :]
        wid = lax.axis_index("s") * 2 + lax.axis_index("c")
        base = wid * rows_per_w
        n_rounds = n_chunks // _NBUF

        for k in range(_NBUF - 1):
            pltpu.async_copy(
                x_hbm.at[pl.ds(base + k * _CH, _CH)], idxs[k], isems[k]
            )

        pltpu.sync_copy(emb_hbm, emb_v)

        @plsc.parallel_loop(0, 19)
        def tab_row(v):
            _fill_table_row(tab_v, emb_v, v, v * 6, 0)

        _fill_table_row(tab_v, emb_v, 19, 110, 4)
        _fill_table_row(tab_v, emb_v, 20, 110, 10)

        def rnd(p, carry):
            for k in range(_NBUF):
                idx_v, out_v, osem, isem = idxs[k], outs[k], osems[k], isems[k]
                c = p * _NBUF + k
                b0 = base + c * _CH

                pltpu.make_async_copy(
                    x_hbm.at[pl.ds(b0, _CH)], idx_v, isem
                ).wait()

                pk = (k + _NBUF - 1) % _NBUF

                @pl.when(c + _NBUF - 1 < n_chunks)
                def _prefetch():
                    nb0 = base + (c + _NBUF - 1) * _CH
                    pltpu.async_copy(
                        x_hbm.at[pl.ds(nb0, _CH)], idxs[pk], isems[pk]
                    )

                @pl.when(p > 0)
                def _drain():
                    pltpu.make_async_copy(
                        out_v, out_hbm.at[pl.ds(b0, _CH)], osem
                    ).wait()

                @plsc.parallel_loop(0, _CH // 16, unroll=2)
                def grp(g):
                    bv = idx_v[pl.ds(g * 16, 16)]
                    for j in range(16):
                        b = bv[j]
                        i = g * 16 + j
                        for o in (0, 16, 32, 48, 64, 80, 96, 107):
                            out_v[i, pl.ds(o, 16)] = tab_v[b, pl.ds(o, 16)]

                pltpu.async_copy(out_v, out_hbm.at[pl.ds(b0, _CH)], osem)
            return carry

        lax.fori_loop(0, n_rounds, rnd, 0)
        for k in range(_NBUF):
            pltpu.make_async_copy(
                outs[k], out_hbm.at[pl.ds(base, _CH)], osems[k]
            ).wait()

    return sc_gather


def kernel(x, embedding):
    x_flat = x.reshape(-1)
    emb_flat = embedding.reshape(-1)
    return _build_sc_gather(x_flat.shape[0])(x_flat, emb_flat)


# speedup vs baseline: 1.7617x; 1.1102x over previous
import functools

import jax
import jax.numpy as jnp
from jax import lax
from jax.experimental import pallas as pl
from jax.experimental.pallas import tpu as pltpu
from jax.experimental.pallas import tpu_sc as plsc

_D = 123
_DP = 128
_NW = 32
_CH = 320
_NBUF = 2


def _fill_table_row(tab_v, emb_v, v, off, sh):
    ev = emb_v[pl.ds(off, 16)]
    h, vol, ch = ev[sh], ev[sh + 1], ev[sh + 2]
    p3, p4, p5 = ev[sh + 3], ev[sh + 4], ev[sh + 5]
    for g in range(8):
        l = lax.iota(jnp.int32, 16) + g * 16
        lf = l.astype(jnp.float32)
        in0 = l < 90
        in1 = l < 112
        in2 = l < 120
        mu = jnp.where(
            in0,
            -4.5 + lf * (9.0 / 89.0),
            jnp.where(
                in1,
                (lf - 90.0) * (2.2 / 21.0),
                -1.0 + (lf - 112.0) * (2.0 / 7.0),
            ),
        )
        val = jnp.where(in0, h, jnp.where(in1, vol, ch))
        inv = jnp.where(in1, 10.0, 4.0)
        d = (val - mu) * inv
        r = jnp.exp(-(d * d))
        pv = jnp.where(l == 120, p3, jnp.where(l == 121, p4, p5))
        s = 1.0 / (1.0 + jnp.exp(3.0 - 6.0 * pv))
        tab_v[v, pl.ds(g * 16, 16)] = jnp.where(in2, r, s)


def _build_sc_gather(n_rows):
    rows_per_w = n_rows // _NW
    n_chunks = rows_per_w // _CH
    mesh = plsc.VectorSubcoreMesh(core_axis_name="c", subcore_axis_name="s")

    @functools.partial(
        pl.kernel,
        mesh=mesh,
        out_type=jax.ShapeDtypeStruct((n_rows, _D), jnp.float32),
        scratch_types=(
            [pltpu.VMEM((21, _DP), jnp.float32)]
            + [pltpu.VMEM((126,), jnp.float32)]
            + [pltpu.VMEM((_CH,), jnp.int32) for _ in range(_NBUF)]
            + [pltpu.VMEM((_CH, _D), jnp.float32) for _ in range(_NBUF)]
            + [pltpu.SemaphoreType.DMA for _ in range(2 * _NBUF)]
        ),
    )
    def sc_gather(x_hbm, emb_hbm, out_hbm, tab_v, emb_v, *bufs_flat):
        idxs = bufs_flat[:_NBUF]
        outs = bufs_flat[_NBUF : 2 * _NBUF]
        osems = bufs_flat[2 * _NBUF : 3 * _NBUF]
        isems = bufs_flat[3 * _NBUF :]
        wid = lax.axis_index("s") * 2 + lax.axis_index("c")
        base = wid * rows_per_w
        n_rounds = n_chunks // _NBUF

        for k in range(_NBUF - 1):
            pltpu.async_copy(
                x_hbm.at[pl.ds(base + k * _CH, _CH)], idxs[k], isems[k]
            )

        pltpu.sync_copy(emb_hbm, emb_v)

        @plsc.parallel_loop(0, 19)
        def tab_row(v):
            _fill_table_row(tab_v, emb_v, v, v * 6, 0)

        _fill_table_row(tab_v, emb_v, 19, 110, 4)
        _fill_table_row(tab_v, emb_v, 20, 110, 10)

        def rnd(p, carry):
            for k in range(_NBUF):
                idx_v, out_v, osem, isem = idxs[k], outs[k], osems[k], isems[k]
                c = p * _NBUF + k
                b0 = base + c * _CH

                pltpu.make_async_copy(
                    x_hbm.at[pl.ds(b0, _CH)], idx_v, isem
                ).wait()

                pk = (k + _NBUF - 1) % _NBUF

                @pl.when(c + _NBUF - 1 < n_chunks)
                def _prefetch():
                    nb0 = base + (c + _NBUF - 1) * _CH
                    pltpu.async_copy(
                        x_hbm.at[pl.ds(nb0, _CH)], idxs[pk], isems[pk]
                    )

                @pl.when(p > 0)
                def _drain():
                    pltpu.make_async_copy(
                        out_v, out_hbm.at[pl.ds(b0, _CH)], osem
                    ).wait()

                @plsc.parallel_loop(0, _CH // 16, unroll=1)
                def grp(g):
                    bv = idx_v[pl.ds(g * 16, 16)]
                    for j in range(16):
                        b = bv[j]
                        i = g * 16 + j
                        for o in (0, 16, 32, 48, 64, 80, 96, 107):
                            out_v[i, pl.ds(o, 16)] = tab_v[b, pl.ds(o, 16)]

                pltpu.async_copy(out_v, out_hbm.at[pl.ds(b0, _CH)], osem)
            return carry

        lax.fori_loop(0, n_rounds, rnd, 0)
        for k in range(_NBUF):
            pltpu.make_async_copy(
                outs[k], out_hbm.at[pl.ds(base, _CH)], osems[k]
            ).wait()

    return sc_gather


def kernel(x, embedding):
    x_flat = x.reshape(-1)
    emb_flat = embedding.reshape(-1)
    return _build_sc_gather(x_flat.shape[0])(x_flat, emb_flat)
